# baseline (speedup 1.0000x reference)
import functools

import jax
import jax.numpy as jnp
import numpy as np
from jax import lax
from jax.experimental import pallas as pl
from jax.experimental.pallas import tpu as pltpu
from jax.experimental.pallas import tpu_sc as plsc

NC = 2
NS = 16
CHUNK = 125

_LOG2 = float(np.log(2.0))


def _k1_body(x_ref, w_ref, o_ref):
    o_ref[...] = jnp.dot(x_ref[...], w_ref[...], preferred_element_type=jnp.float32)


def _matmul_xw1(x, W1, block_rows):
    n, feat = x.shape
    emb = W1.shape[1]
    grid = (n // block_rows,)
    return pl.pallas_call(
        _k1_body,
        grid=grid,
        in_specs=[
            pl.BlockSpec((block_rows, feat), lambda i: (i, 0)),
            pl.BlockSpec((feat, emb), lambda i: (0, 0)),
        ],
        out_specs=pl.BlockSpec((block_rows, emb), lambda i: (i, 0)),
        out_shape=jax.ShapeDtypeStruct((n, emb), jnp.float32),
    )(x, W1)


def _sc_scatter(y, src2, dst2, zeros_tile):
    n_acc, emb = y.shape
    ch = src2.shape[0]
    ch_t = ch // NC // NS
    rows_per_tile = n_acc // NS
    mesh = plsc.VectorSubcoreMesh(
        core_axis_name="c", subcore_axis_name="s", num_cores=NC, num_subcores=NS
    )

    @functools.partial(
        pl.kernel,
        out_type=jax.ShapeDtypeStruct((NC, n_acc, emb), jnp.float32),
        mesh=mesh,
        compiler_params=pltpu.CompilerParams(use_tc_tiling_on_sc=False),
        scratch_types=[
            pltpu.VMEM((ch_t, CHUNK), jnp.int32),
            pltpu.VMEM((ch_t, CHUNK), jnp.int32),
            pltpu.VMEM((CHUNK, emb), jnp.float32),
            pltpu.VMEM_SHARED((n_acc, emb), jnp.float32),
            pltpu.SemaphoreType.DMA,
        ],
    )
    def k(y_hbm, src_hbm, dst_hbm, z_hbm, out_hbm, src_v, dst_v, rows0, acc,
          sem0):
        c = lax.axis_index("c")
        s = lax.axis_index("s")
        t = c * NS + s
        base = s * rows_per_tile

        pltpu.sync_copy(z_hbm, acc.at[pl.ds(base, rows_per_tile)])
        pltpu.sync_copy(src_hbm.at[pl.ds(t * ch_t, ch_t)], src_v)
        pltpu.sync_copy(dst_hbm.at[pl.ds(t * ch_t, ch_t)], dst_v)
        plsc.subcore_barrier()

        def body(j, _):
            pltpu.async_copy(y_hbm.at[src_v.at[j]], rows0, sem0).wait()
            pltpu.sync_copy(rows0, acc.at[dst_v.at[j]], add=True)
            return 0

        lax.fori_loop(0, ch_t, body, 0, unroll=False)
        plsc.subcore_barrier()

        pltpu.sync_copy(
            acc.at[pl.ds(base, rows_per_tile)],
            out_hbm.at[c].at[pl.ds(base, rows_per_tile)],
        )

    return k(y, src2, dst2, zeros_tile)


def _ff_block(z, Wa, ba, Wb, bb, Wc, bc, Ws, bs):
    t = jnp.maximum(jnp.dot(z, Wa, preferred_element_type=jnp.float32) + ba, 0.0)
    t = jnp.maximum(jnp.dot(t, Wb, preferred_element_type=jnp.float32) + bb, 0.0)
    t = jnp.maximum(jnp.dot(t, Wc, preferred_element_type=jnp.float32) + bc, 0.0)
    return t + jnp.dot(z, Ws, preferred_element_type=jnp.float32) + bs


def _k23_body(n_nodes, n_graphs, block_rows,
              p0_ref, p1_ref, y_ref, gid_ref, w2_ref, b2_ref, b1_ref,
              lwa_ref, lba_ref, lwb_ref, lbb_ref, lwc_ref, lbc_ref, lws_ref,
              lbs_ref, gwa_ref, gba_ref, gwb_ref, gbb_ref, gwc_ref, gbc_ref,
              gws_ref, gbs_ref, out_ref, lenc_scr, gh_scr, genc_scr):
    p = pl.program_id(0)
    i = pl.program_id(1)

    @pl.when(p == 0)
    def _():
        h1 = jnp.maximum(p0_ref[...] + p1_ref[...] + y_ref[...] + b1_ref[...],
                         0.0)
        h2 = (jnp.dot(h1, w2_ref[...], preferred_element_type=jnp.float32)
              + b2_ref[...])
        lh = jnp.maximum(h2, 0.0)
        lenc_scr[pl.ds(i * block_rows, block_rows), :] = _ff_block(
            lh, lwa_ref[...], lba_ref[...], lwb_ref[...], lbb_ref[...],
            lwc_ref[...], lbc_ref[...], lws_ref[...], lbs_ref[...])
        gid = gid_ref[...]
        onehot = (gid == lax.broadcasted_iota(
            jnp.int32, (block_rows, n_graphs), 1)).astype(jnp.float32)
        contrib = lax.dot_general(onehot, lh, (((0,), (0,)), ((), ())),
                                  preferred_element_type=jnp.float32)

        @pl.when(i == 0)
        def _():
            gh_scr[...] = contrib

        @pl.when(i != 0)
        def _():
            gh_scr[...] += contrib

    @pl.when(p == 1)
    def _():
        @pl.when(i == 0)
        def _():
            genc_scr[...] = _ff_block(
                gh_scr[...], gwa_ref[...], gba_ref[...], gwb_ref[...],
                gbb_ref[...], gwc_ref[...], gbc_ref[...], gws_ref[...],
                gbs_ref[...])
            out_ref[0, 0] = 0.0

        lenc = lenc_scr[pl.ds(i * block_rows, block_rows), :]
        res = lax.dot_general(lenc, genc_scr[...], (((1,), (1,)), ((), ())),
                              preferred_element_type=jnp.float32)
        gid = gid_ref[...]
        cols = lax.broadcasted_iota(jnp.int32, (block_rows, n_graphs), 1)
        pos = gid == cols
        soft = jnp.log1p(jnp.exp(-jnp.abs(res)))
        sp_m = jnp.maximum(-res, 0.0) + soft
        sp_p = jnp.maximum(res, 0.0) + soft
        neg_c = jnp.where(pos, 0.0, sp_p - _LOG2)
        pos_c = jnp.where(pos, _LOG2 - sp_m, 0.0)
        val = (jnp.sum(neg_c) / (n_nodes * (n_graphs - 1))
               - jnp.sum(pos_c) / n_nodes)
        out_ref[0, 0] += val


def _k23(p0, p1, y, gid2, W2, b2r, b1r, lWa, lbar, lWb, lbbr, lWc, lbcr,
         lWs, lbsr, gWa, gbar, gWb, gbbr, gWc, gbcr, gWs, gbsr,
         n_nodes, n_graphs, block_rows):
    n, emb = y.shape
    nb = n // block_rows
    grid = (2, nb)
    row_spec = pl.BlockSpec((block_rows, emb), lambda p, i: (i * (1 - p), 0))
    gid_spec = pl.BlockSpec((block_rows, 1), lambda p, i: (i, 0))
    w_spec = pl.BlockSpec((emb, emb), lambda p, i: (0, 0))
    b_spec = pl.BlockSpec((1, emb), lambda p, i: (0, 0))
    return pl.pallas_call(
        functools.partial(_k23_body, n_nodes, n_graphs, block_rows),
        grid=grid,
        in_specs=[
            row_spec, row_spec, row_spec, gid_spec,
            w_spec, b_spec, b_spec,
            w_spec, b_spec, w_spec, b_spec, w_spec, b_spec, w_spec, b_spec,
            w_spec, b_spec, w_spec, b_spec, w_spec, b_spec, w_spec, b_spec,
        ],
        out_specs=pl.BlockSpec(memory_space=pltpu.SMEM),
        out_shape=jax.ShapeDtypeStruct((1, 1), jnp.float32),
        scratch_shapes=[
            pltpu.VMEM((n, emb), jnp.float32),
            pltpu.VMEM((n_graphs, emb), jnp.float32),
            pltpu.VMEM((n_graphs, emb), jnp.float32),
        ],
    )(p0, p1, y, gid2, W2, b2r, b1r, lWa, lbar, lWb, lbbr, lWc, lbcr,
      lWs, lbsr, gWa, gbar, gWb, gbbr, gWc, gbcr, gWs, gbsr)


def kernel(x, edge_index, graph_id, W1, b1, W2, b2, lWa, lba, lWb, lbb, lWc,
           lbc, lWs, lbs, gWa, gba, gWb, gbb, gWc, gbc, gWs, gbs):
    n_nodes, feat = x.shape
    emb = W1.shape[1]
    n_graphs = 128
    n_edges = edge_index.shape[1]

    block_rows = 2000

    n_chunks = n_edges // CHUNK
    src2 = edge_index[0].reshape(n_chunks, CHUNK)
    dst2 = edge_index[1].reshape(n_chunks, CHUNK)

    y = _matmul_xw1(x, W1, block_rows)
    zeros_tile = jnp.zeros((n_nodes // NS, emb), jnp.float32)
    partials = _sc_scatter(y, src2, dst2, zeros_tile)

    gid2 = graph_id[:, None]
    out = _k23(partials[0], partials[1], y, gid2, W2, b2[None, :], b1[None, :],
               lWa, lba[None, :], lWb, lbb[None, :], lWc, lbc[None, :],
               lWs, lbs[None, :], gWa, gba[None, :], gWb, gbb[None, :],
               gWc, gbc[None, :], gWs, gbs[None, :], n_nodes, n_graphs,
               block_rows)
    return out[0, 0]


# speedup vs baseline: 1.0869x; 1.0869x over previous
import functools

import jax
import jax.numpy as jnp
import numpy as np
from jax import lax
from jax.experimental import pallas as pl
from jax.experimental.pallas import tpu as pltpu
from jax.experimental.pallas import tpu_sc as plsc

NC = 2
NS = 16
CHUNK = 125

_LOG2 = float(np.log(2.0))


def _k1_body(x_ref, w_ref, o_ref):
    o_ref[...] = jnp.dot(x_ref[...], w_ref[...], preferred_element_type=jnp.float32)


def _matmul_xw1(x, W1, block_rows):
    n, feat = x.shape
    emb = W1.shape[1]
    grid = (n // block_rows,)
    return pl.pallas_call(
        _k1_body,
        grid=grid,
        in_specs=[
            pl.BlockSpec((block_rows, feat), lambda i: (i, 0)),
            pl.BlockSpec((feat, emb), lambda i: (0, 0)),
        ],
        out_specs=pl.BlockSpec((block_rows, emb), lambda i: (i, 0)),
        out_shape=jax.ShapeDtypeStruct((n, emb), jnp.float32),
    )(x, W1)


def _sc_scatter(y, ei3, zeros_acc):
    n_acc, emb = y.shape
    ch = ei3.shape[1]
    ch_t = ch // NC // NS
    rows_per_tile = n_acc // NS
    mesh = plsc.VectorSubcoreMesh(
        core_axis_name="c", subcore_axis_name="s", num_cores=NC, num_subcores=NS
    )

    @functools.partial(
        pl.kernel,
        out_type=jax.ShapeDtypeStruct((NC, n_acc, emb), jnp.float32),
        mesh=mesh,
        compiler_params=pltpu.CompilerParams(use_tc_tiling_on_sc=False),
        scratch_types=[
            pltpu.VMEM((ch_t, CHUNK), jnp.int32),
            pltpu.VMEM((ch_t, CHUNK), jnp.int32),
            pltpu.VMEM((CHUNK, emb), jnp.float32),
            pltpu.VMEM_SHARED((n_acc, emb), jnp.float32),
            pltpu.SemaphoreType.DMA,
        ],
    )
    def k(y_hbm, ei_hbm, z_hbm, out_hbm, src_v, dst_v, rows0, acc, sem0):
        c = lax.axis_index("c")
        s = lax.axis_index("s")
        t = c * NS + s
        base = s * rows_per_tile

        @pl.when(s == 0)
        def _():
            pltpu.sync_copy(z_hbm, acc)

        pltpu.sync_copy(ei_hbm.at[0].at[pl.ds(t * ch_t, ch_t)], src_v)
        pltpu.sync_copy(ei_hbm.at[1].at[pl.ds(t * ch_t, ch_t)], dst_v)
        plsc.subcore_barrier()

        def body(j, _):
            pltpu.async_copy(y_hbm.at[src_v.at[j]], rows0, sem0).wait()
            pltpu.sync_copy(rows0, acc.at[dst_v.at[j]], add=True)
            return 0

        lax.fori_loop(0, ch_t, body, 0, unroll=False)
        plsc.subcore_barrier()

        pltpu.sync_copy(
            acc.at[pl.ds(base, rows_per_tile)],
            out_hbm.at[c].at[pl.ds(base, rows_per_tile)],
        )

    return k(y, ei3, zeros_acc)


def _ff_block(z, Wa, ba, Wb, bb, Wc, bc, Ws, bs):
    t = jnp.maximum(jnp.dot(z, Wa, preferred_element_type=jnp.float32) + ba, 0.0)
    t = jnp.maximum(jnp.dot(t, Wb, preferred_element_type=jnp.float32) + bb, 0.0)
    t = jnp.maximum(jnp.dot(t, Wc, preferred_element_type=jnp.float32) + bc, 0.0)
    return t + jnp.dot(z, Ws, preferred_element_type=jnp.float32) + bs


def _k23_body(n_nodes, n_graphs, block_rows,
              p0_ref, p1_ref, y_ref, gid_ref, w2_ref, b2_ref, b1_ref,
              lwa_ref, lba_ref, lwb_ref, lbb_ref, lwc_ref, lbc_ref, lws_ref,
              lbs_ref, gwa_ref, gba_ref, gwb_ref, gbb_ref, gwc_ref, gbc_ref,
              gws_ref, gbs_ref, out_ref, lenc_scr, gh_scr, genc_scr):
    p = pl.program_id(0)
    i = pl.program_id(1)

    @pl.when(p == 0)
    def _():
        h1 = jnp.maximum(p0_ref[...] + p1_ref[...] + y_ref[...] + b1_ref[...],
                         0.0)
        h2 = (jnp.dot(h1, w2_ref[...], preferred_element_type=jnp.float32)
              + b2_ref[...])
        lh = jnp.maximum(h2, 0.0)
        lenc_scr[pl.ds(i * block_rows, block_rows), :] = _ff_block(
            lh, lwa_ref[...], lba_ref[...], lwb_ref[...], lbb_ref[...],
            lwc_ref[...], lbc_ref[...], lws_ref[...], lbs_ref[...])
        gid = gid_ref[...]
        onehot = (gid == lax.broadcasted_iota(
            jnp.int32, (block_rows, n_graphs), 1)).astype(jnp.float32)
        contrib = lax.dot_general(onehot, lh, (((0,), (0,)), ((), ())),
                                  preferred_element_type=jnp.float32)

        @pl.when(i == 0)
        def _():
            gh_scr[...] = contrib

        @pl.when(i != 0)
        def _():
            gh_scr[...] += contrib

    @pl.when(p == 1)
    def _():
        @pl.when(i == 0)
        def _():
            genc_scr[...] = _ff_block(
                gh_scr[...], gwa_ref[...], gba_ref[...], gwb_ref[...],
                gbb_ref[...], gwc_ref[...], gbc_ref[...], gws_ref[...],
                gbs_ref[...])
            out_ref[0, 0] = 0.0

        lenc = lenc_scr[pl.ds(i * block_rows, block_rows), :]
        res = lax.dot_general(lenc, genc_scr[...], (((1,), (1,)), ((), ())),
                              preferred_element_type=jnp.float32)
        gid = gid_ref[...]
        cols = lax.broadcasted_iota(jnp.int32, (block_rows, n_graphs), 1)
        pos = gid == cols
        soft = jnp.log1p(jnp.exp(-jnp.abs(res)))
        sp_m = jnp.maximum(-res, 0.0) + soft
        sp_p = jnp.maximum(res, 0.0) + soft
        neg_c = jnp.where(pos, 0.0, sp_p - _LOG2)
        pos_c = jnp.where(pos, _LOG2 - sp_m, 0.0)
        val = (jnp.sum(neg_c) / (n_nodes * (n_graphs - 1))
               - jnp.sum(pos_c) / n_nodes)
        out_ref[0, 0] += val


def _k23(p0, p1, y, gid2, W2, b2r, b1r, lWa, lbar, lWb, lbbr, lWc, lbcr,
         lWs, lbsr, gWa, gbar, gWb, gbbr, gWc, gbcr, gWs, gbsr,
         n_nodes, n_graphs, block_rows):
    n, emb = y.shape
    nb = n // block_rows
    grid = (2, nb)
    row_spec = pl.BlockSpec((block_rows, emb), lambda p, i: (i * (1 - p), 0))
    gid_spec = pl.BlockSpec((block_rows, 1), lambda p, i: (i, 0))
    w_spec = pl.BlockSpec((emb, emb), lambda p, i: (0, 0))
    b_spec = pl.BlockSpec((1, emb), lambda p, i: (0, 0))
    return pl.pallas_call(
        functools.partial(_k23_body, n_nodes, n_graphs, block_rows),
        grid=grid,
        in_specs=[
            row_spec, row_spec, row_spec, gid_spec,
            w_spec, b_spec, b_spec,
            w_spec, b_spec, w_spec, b_spec, w_spec, b_spec, w_spec, b_spec,
            w_spec, b_spec, w_spec, b_spec, w_spec, b_spec, w_spec, b_spec,
        ],
        out_specs=pl.BlockSpec(memory_space=pltpu.SMEM),
        out_shape=jax.ShapeDtypeStruct((1, 1), jnp.float32),
        scratch_shapes=[
            pltpu.VMEM((n, emb), jnp.float32),
            pltpu.VMEM((n_graphs, emb), jnp.float32),
            pltpu.VMEM((n_graphs, emb), jnp.float32),
        ],
    )(p0, p1, y, gid2, W2, b2r, b1r, lWa, lbar, lWb, lbbr, lWc, lbcr,
      lWs, lbsr, gWa, gbar, gWb, gbbr, gWc, gbcr, gWs, gbsr)


def kernel(x, edge_index, graph_id, W1, b1, W2, b2, lWa, lba, lWb, lbb, lWc,
           lbc, lWs, lbs, gWa, gba, gWb, gbb, gWc, gbc, gWs, gbs):
    n_nodes, feat = x.shape
    emb = W1.shape[1]
    n_graphs = 128
    n_edges = edge_index.shape[1]

    block_rows = 2000

    n_chunks = n_edges // CHUNK
    ei3 = edge_index.reshape(2, n_chunks, CHUNK)

    y = _matmul_xw1(x, W1, block_rows)
    zeros_acc = jnp.zeros((n_nodes, emb), jnp.float32)
    partials = _sc_scatter(y, ei3, zeros_acc)

    gid2 = graph_id[:, None]
    out = _k23(partials[0], partials[1], y, gid2, W2, b2[None, :], b1[None, :],
               lWa, lba[None, :], lWb, lbb[None, :], lWc, lbc[None, :],
               lWs, lbs[None, :], gWa, gba[None, :], gWb, gbb[None, :],
               gWc, gbc[None, :], gWs, gbs[None, :], n_nodes, n_graphs,
               block_rows)
    return out[0, 0]


# speedup vs baseline: 1.0957x; 1.0081x over previous
import functools

import jax
import jax.numpy as jnp
import numpy as np
from jax import lax
from jax.experimental import pallas as pl
from jax.experimental.pallas import tpu as pltpu
from jax.experimental.pallas import tpu_sc as plsc

NC = 2
NS = 16
CHUNK = 125

_LOG2 = float(np.log(2.0))


def _k1_body(x_ref, w_ref, o_ref):
    o_ref[...] = jnp.dot(x_ref[...], w_ref[...], preferred_element_type=jnp.float32)


def _matmul_xw1(x, W1, block_rows):
    n, feat = x.shape
    emb = W1.shape[1]
    grid = (n // block_rows,)
    return pl.pallas_call(
        _k1_body,
        grid=grid,
        in_specs=[
            pl.BlockSpec((block_rows, feat), lambda i: (i, 0)),
            pl.BlockSpec((feat, emb), lambda i: (0, 0)),
        ],
        out_specs=pl.BlockSpec((block_rows, emb), lambda i: (i, 0)),
        out_shape=jax.ShapeDtypeStruct((n, emb), jnp.float32),
    )(x, W1)


def _sc_scatter(y_rows_in, ei3, zeros_acc):
    n_acc, emb = y_rows_in.shape
    ch = ei3.shape[1]
    ch_t = ch // NC // NS
    rows_per_tile = n_acc // NS
    mesh = plsc.VectorSubcoreMesh(
        core_axis_name="c", subcore_axis_name="s", num_cores=NC, num_subcores=NS
    )

    @functools.partial(
        pl.kernel,
        out_type=jax.ShapeDtypeStruct((NC, n_acc, emb), jnp.float32),
        mesh=mesh,
        compiler_params=pltpu.CompilerParams(use_tc_tiling_on_sc=False),
        scratch_types=[
            pltpu.VMEM((ch_t, CHUNK), jnp.int32),
            pltpu.VMEM((ch_t, CHUNK), jnp.int32),
            pltpu.VMEM((CHUNK, emb), jnp.float32),
            pltpu.VMEM_SHARED((n_acc, emb), jnp.float32),
            pltpu.SemaphoreType.DMA,
        ],
    )
    def k(y_hbm, ei_hbm, z_hbm, out_hbm, src_v, dst_v, rows0, acc, sem0):
        c = lax.axis_index("c")
        s = lax.axis_index("s")
        t = c * NS + s
        base = s * rows_per_tile
        y_rows = y_hbm
        acc_rows = acc

        @pl.when(s == 0)
        def _():
            pltpu.sync_copy(z_hbm, acc)

        pltpu.sync_copy(ei_hbm.at[0].at[pl.ds(t * ch_t, ch_t)], src_v)
        pltpu.sync_copy(ei_hbm.at[1].at[pl.ds(t * ch_t, ch_t)], dst_v)
        plsc.subcore_barrier()

        def body(j, _):
            pltpu.async_copy(y_rows.at[src_v.at[j]], rows0, sem0).wait()
            pltpu.sync_copy(rows0, acc_rows.at[dst_v.at[j]], add=True)
            return 0

        lax.fori_loop(0, ch_t, body, 0, unroll=False)
        plsc.subcore_barrier()

        pltpu.sync_copy(
            acc.at[pl.ds(base, rows_per_tile)],
            out_hbm.at[c].at[pl.ds(base, rows_per_tile)],
        )

    return k(y_rows_in, ei3, zeros_acc)


def _ff_block(z, Wa, ba, Wb, bb, Wc, bc, Ws, bs):
    t = jnp.maximum(jnp.dot(z, Wa, preferred_element_type=jnp.float32) + ba, 0.0)
    t = jnp.maximum(jnp.dot(t, Wb, preferred_element_type=jnp.float32) + bb, 0.0)
    t = jnp.maximum(jnp.dot(t, Wc, preferred_element_type=jnp.float32) + bc, 0.0)
    return t + jnp.dot(z, Ws, preferred_element_type=jnp.float32) + bs


def _k23_body(n_nodes, n_graphs, half_rows, emb,
              p0_ref, p1_ref, y_ref, gid_ref, w2_ref, b2_ref, b1_ref,
              lwa_ref, lba_ref, lwb_ref, lbb_ref, lwc_ref, lbc_ref, lws_ref,
              lbs_ref, gwa_ref, gba_ref, gwb_ref, gbb_ref, gwc_ref, gbc_ref,
              gws_ref, gbs_ref, out_ref, lenc_scr, gh_scr, genc_scr):
    p = pl.program_id(0)
    i = pl.program_id(1)

    @pl.when(p == 0)
    def _():
        h1 = jnp.maximum(
            p0_ref[...] + p1_ref[...] + y_ref[...] + b1_ref[...], 0.0)
        h2 = (jnp.dot(h1, w2_ref[...], preferred_element_type=jnp.float32)
              + b2_ref[...])
        lh = jnp.maximum(h2, 0.0)
        lenc_scr[pl.ds(i * half_rows, half_rows), :] = _ff_block(
            lh, lwa_ref[...], lba_ref[...], lwb_ref[...], lbb_ref[...],
            lwc_ref[...], lbc_ref[...], lws_ref[...], lbs_ref[...])
        gid = gid_ref[...]
        iot = lax.broadcasted_iota(jnp.int32, (half_rows, n_graphs), 1)
        oh_e = (gid[:, 0:1] == iot).astype(jnp.float32)
        oh_o = (gid[:, 1:2] == iot).astype(jnp.float32)
        contrib = (
            lax.dot_general(oh_e, lh[:, :emb], (((0,), (0,)), ((), ())),
                            preferred_element_type=jnp.float32)
            + lax.dot_general(oh_o, lh[:, emb:], (((0,), (0,)), ((), ())),
                              preferred_element_type=jnp.float32))

        @pl.when(i == 0)
        def _():
            gh_scr[...] = contrib

        @pl.when(i != 0)
        def _():
            gh_scr[...] += contrib

    @pl.when(p == 1)
    def _():
        @pl.when(i == 0)
        def _():
            genc_scr[...] = _ff_block(
                gh_scr[...], gwa_ref[...], gba_ref[...], gwb_ref[...],
                gbb_ref[...], gwc_ref[...], gbc_ref[...], gws_ref[...],
                gbs_ref[...])
            out_ref[0, 0] = 0.0

        lenc = lenc_scr[pl.ds(i * half_rows, half_rows), :]
        gid = gid_ref[...]
        genc = genc_scr[...]
        iot = lax.broadcasted_iota(jnp.int32, (half_rows, n_graphs), 1)
        acc = jnp.zeros((), jnp.float32)
        for lo, hi, col in ((0, emb, 0), (emb, 2 * emb, 1)):
            res = lax.dot_general(lenc[:, lo:hi], genc,
                                  (((1,), (1,)), ((), ())),
                                  preferred_element_type=jnp.float32)
            pos = gid[:, col:col + 1] == iot
            soft = jnp.log1p(jnp.exp(-jnp.abs(res)))
            sp_m = jnp.maximum(-res, 0.0) + soft
            sp_p = jnp.maximum(res, 0.0) + soft
            neg_c = jnp.where(pos, 0.0, sp_p - _LOG2)
            pos_c = jnp.where(pos, _LOG2 - sp_m, 0.0)
            acc += (jnp.sum(neg_c) / (n_nodes * (n_graphs - 1))
                    - jnp.sum(pos_c) / n_nodes)
        out_ref[0, 0] += acc


def _k23(p0, p1, y, gid2, W2d, b2d, b1d, lWad, lbad, lWbd, lbbd, lWcd, lbcd,
         lWsd, lbsd, gWa, gbar, gWb, gbbr, gWc, gbcr, gWs, gbsr,
         n_nodes, n_graphs, block_rows):
    half, twoemb = y.shape
    emb = twoemb // 2
    half_rows = block_rows // 2
    nb = half // half_rows
    grid = (2, nb)
    row_spec = pl.BlockSpec((half_rows, twoemb), lambda p, i: (i * (1 - p), 0))
    gid_spec = pl.BlockSpec((half_rows, 2), lambda p, i: (i, 0))
    wd_spec = pl.BlockSpec((twoemb, twoemb), lambda p, i: (0, 0))
    bd_spec = pl.BlockSpec((1, twoemb), lambda p, i: (0, 0))
    w_spec = pl.BlockSpec((emb, emb), lambda p, i: (0, 0))
    b_spec = pl.BlockSpec((1, emb), lambda p, i: (0, 0))
    return pl.pallas_call(
        functools.partial(_k23_body, n_nodes, n_graphs, half_rows, emb),
        grid=grid,
        in_specs=[
            row_spec, row_spec, row_spec, gid_spec,
            wd_spec, bd_spec, bd_spec,
            wd_spec, bd_spec, wd_spec, bd_spec, wd_spec, bd_spec, wd_spec,
            bd_spec,
            w_spec, b_spec, w_spec, b_spec, w_spec, b_spec, w_spec, b_spec,
        ],
        out_specs=pl.BlockSpec(memory_space=pltpu.SMEM),
        out_shape=jax.ShapeDtypeStruct((1, 1), jnp.float32),
        scratch_shapes=[
            pltpu.VMEM((half, twoemb), jnp.float32),
            pltpu.VMEM((n_graphs, emb), jnp.float32),
            pltpu.VMEM((n_graphs, emb), jnp.float32),
        ],
    )(p0, p1, y, gid2, W2d, b2d, b1d, lWad, lbad, lWbd, lbbd, lWcd, lbcd,
      lWsd, lbsd, gWa, gbar, gWb, gbbr, gWc, gbcr, gWs, gbsr)


def kernel(x, edge_index, graph_id, W1, b1, W2, b2, lWa, lba, lWb, lbb, lWc,
           lbc, lWs, lbs, gWa, gba, gWb, gbb, gWc, gbc, gWs, gbs):
    n_nodes, feat = x.shape
    emb = W1.shape[1]
    n_graphs = 128
    n_edges = edge_index.shape[1]

    block_rows = 2000

    n_chunks = n_edges // CHUNK
    ei3 = edge_index.reshape(2, n_chunks, CHUNK)

    y = _matmul_xw1(x, W1, block_rows)
    zeros_acc = jnp.zeros((n_nodes, emb), jnp.float32)
    partials = _sc_scatter(y, ei3, zeros_acc)

    half = n_nodes // 2
    y_p = y.reshape(half, 2 * emb)
    pp = partials.reshape(NC, half, 2 * emb)
    gid2 = graph_id.reshape(half, 2)

    def bd(w):
        z = jnp.zeros_like(w)
        return jnp.concatenate(
            [jnp.concatenate([w, z], 1), jnp.concatenate([z, w], 1)], 0)

    def b2x(b):
        return jnp.concatenate([b, b])[None, :]

    out = _k23(pp[0], pp[1], y_p, gid2, bd(W2), b2x(b2), b2x(b1),
               bd(lWa), b2x(lba), bd(lWb), b2x(lbb), bd(lWc), b2x(lbc),
               bd(lWs), b2x(lbs), gWa, gba[None, :], gWb, gbb[None, :],
               gWc, gbc[None, :], gWs, gbs[None, :], n_nodes, n_graphs,
               block_rows)
    return out[0, 0]


# speedup vs baseline: 1.1096x; 1.0127x over previous
import functools

import jax
import jax.numpy as jnp
import numpy as np
from jax import lax
from jax.experimental import pallas as pl
from jax.experimental.pallas import tpu as pltpu
from jax.experimental.pallas import tpu_sc as plsc

NC = 2
NS = 16
CHUNK = 125

_LOG2 = float(np.log(2.0))


def _k1_body(x_ref, w_ref, o_ref):
    o_ref[...] = jnp.dot(x_ref[...], w_ref[...], preferred_element_type=jnp.float32)


def _matmul_xw1(x, W1, block_rows):
    n, feat = x.shape
    emb = W1.shape[1]
    grid = (n // block_rows,)
    return pl.pallas_call(
        _k1_body,
        grid=grid,
        in_specs=[
            pl.BlockSpec((block_rows, feat), lambda i: (i, 0)),
            pl.BlockSpec((feat, emb), lambda i: (0, 0)),
        ],
        out_specs=pl.BlockSpec((block_rows, emb), lambda i: (i, 0)),
        out_shape=jax.ShapeDtypeStruct((n, emb), jnp.float32),
    )(x, W1)


def _sc_scatter(y_rows_in, ei3, zeros_acc):
    n_acc, emb = y_rows_in.shape
    ch = ei3.shape[1]
    ch_t = ch // NC // NS
    rows_per_tile = n_acc // NS
    mesh = plsc.VectorSubcoreMesh(
        core_axis_name="c", subcore_axis_name="s", num_cores=NC, num_subcores=NS
    )

    @functools.partial(
        pl.kernel,
        out_type=jax.ShapeDtypeStruct((NC, n_acc, emb), jnp.float32),
        mesh=mesh,
        compiler_params=pltpu.CompilerParams(use_tc_tiling_on_sc=False),
        scratch_types=[
            pltpu.VMEM((ch_t, CHUNK), jnp.int32),
            pltpu.VMEM((ch_t, CHUNK), jnp.int32),
            pltpu.VMEM((CHUNK, emb), jnp.float32),
            pltpu.VMEM_SHARED((n_acc, emb), jnp.float32),
            pltpu.SemaphoreType.DMA,
        ],
    )
    def k(y_hbm, ei_hbm, z_hbm, out_hbm, src_v, dst_v, rows0, acc, sem0):
        c = lax.axis_index("c")
        s = lax.axis_index("s")
        t = c * NS + s
        base = s * rows_per_tile
        y_rows = y_hbm
        acc_rows = acc

        @pl.when(s == 0)
        def _():
            pltpu.sync_copy(z_hbm, acc)

        pltpu.sync_copy(ei_hbm.at[0].at[pl.ds(t * ch_t, ch_t)], src_v)
        pltpu.sync_copy(ei_hbm.at[1].at[pl.ds(t * ch_t, ch_t)], dst_v)
        plsc.subcore_barrier()

        def body(j, _):
            pltpu.async_copy(y_rows.at[src_v.at[j]], rows0, sem0).wait()
            pltpu.sync_copy(rows0, acc_rows.at[dst_v.at[j]], add=True)
            return 0

        lax.fori_loop(0, ch_t, body, 0, unroll=False)
        plsc.subcore_barrier()

        pltpu.sync_copy(
            acc.at[pl.ds(base, rows_per_tile)],
            out_hbm.at[c].at[pl.ds(base, rows_per_tile)],
        )

    return k(y_rows_in, ei3, zeros_acc)


def _ff_block(z, Wa, ba, Wb, bb, Wc, bc, Ws, bs):
    t = jnp.maximum(jnp.dot(z, Wa, preferred_element_type=jnp.float32) + ba, 0.0)
    t = jnp.maximum(jnp.dot(t, Wb, preferred_element_type=jnp.float32) + bb, 0.0)
    t = jnp.maximum(jnp.dot(t, Wc, preferred_element_type=jnp.float32) + bc, 0.0)
    return t + jnp.dot(z, Ws, preferred_element_type=jnp.float32) + bs


def _k23_body(n_nodes, n_graphs, half_rows, emb,
              p0_ref, p1_ref, y_ref, gid_ref, w2_ref, b2_ref, b1_ref,
              lwa_ref, lba_ref, lwb_ref, lbb_ref, lwc_ref, lbc_ref, lws_ref,
              lbs_ref, gwa_ref, gba_ref, gwb_ref, gbb_ref, gwc_ref, gbc_ref,
              gws_ref, gbs_ref, out_ref, lenc_scr, gh_scr, genc_scr):
    p = pl.program_id(0)
    i = pl.program_id(1)

    @pl.when(p == 0)
    def _():
        h1 = jnp.maximum(
            p0_ref[...] + p1_ref[...] + y_ref[...] + b1_ref[...], 0.0)
        h2 = (jnp.dot(h1, w2_ref[...], preferred_element_type=jnp.float32)
              + b2_ref[...])
        lh = jnp.maximum(h2, 0.0)
        lenc_scr[pl.ds(i * half_rows, half_rows), :] = _ff_block(
            lh, lwa_ref[...], lba_ref[...], lwb_ref[...], lbb_ref[...],
            lwc_ref[...], lbc_ref[...], lws_ref[...], lbs_ref[...])
        gid = gid_ref[...]
        iot = lax.broadcasted_iota(jnp.int32, (half_rows, n_graphs), 1)
        oh_e = (gid[:, 0:1] == iot).astype(jnp.float32)
        oh_o = (gid[:, 1:2] == iot).astype(jnp.float32)
        contrib = (
            lax.dot_general(oh_e, lh[:, :emb], (((0,), (0,)), ((), ())),
                            preferred_element_type=jnp.float32)
            + lax.dot_general(oh_o, lh[:, emb:], (((0,), (0,)), ((), ())),
                              preferred_element_type=jnp.float32))

        @pl.when(i == 0)
        def _():
            gh_scr[...] = contrib

        @pl.when(i != 0)
        def _():
            gh_scr[...] += contrib

    @pl.when(p == 1)
    def _():
        @pl.when(i == 0)
        def _():
            genc_scr[...] = _ff_block(
                gh_scr[...], gwa_ref[...], gba_ref[...], gwb_ref[...],
                gbb_ref[...], gwc_ref[...], gbc_ref[...], gws_ref[...],
                gbs_ref[...])
            out_ref[0, 0] = 0.0

        lenc = lenc_scr[pl.ds(i * half_rows, half_rows), :]
        gid = gid_ref[...]
        genc = genc_scr[...]
        iot = lax.broadcasted_iota(jnp.int32, (half_rows, n_graphs), 1)
        acc = jnp.zeros((), jnp.float32)
        for lo, hi, col in ((0, emb, 0), (emb, 2 * emb, 1)):
            res = lax.dot_general(lenc[:, lo:hi], genc,
                                  (((1,), (1,)), ((), ())),
                                  preferred_element_type=jnp.float32)
            pos = gid[:, col:col + 1] == iot
            soft = jnp.log1p(jnp.exp(-jnp.abs(res)))
            sp_m = jnp.maximum(-res, 0.0) + soft
            sp_p = jnp.maximum(res, 0.0) + soft
            neg_c = jnp.where(pos, 0.0, sp_p - _LOG2)
            pos_c = jnp.where(pos, _LOG2 - sp_m, 0.0)
            acc += (jnp.sum(neg_c) / (n_nodes * (n_graphs - 1))
                    - jnp.sum(pos_c) / n_nodes)
        out_ref[0, 0] += acc


def _k23(p0, p1, y, gid2, W2d, b2d, b1d, lWad, lbad, lWbd, lbbd, lWcd, lbcd,
         lWsd, lbsd, gWa, gbar, gWb, gbbr, gWc, gbcr, gWs, gbsr,
         n_nodes, n_graphs, block_rows):
    half, twoemb = y.shape
    emb = twoemb // 2
    half_rows = block_rows // 2
    nb = half // half_rows
    grid = (2, nb)
    row_spec = pl.BlockSpec((half_rows, twoemb), lambda p, i: (i * (1 - p), 0))
    gid_spec = pl.BlockSpec((half_rows, 2), lambda p, i: (i, 0))
    wd_spec = pl.BlockSpec((twoemb, twoemb), lambda p, i: (0, 0))
    bd_spec = pl.BlockSpec((1, twoemb), lambda p, i: (0, 0))
    w_spec = pl.BlockSpec((emb, emb), lambda p, i: (0, 0))
    b_spec = pl.BlockSpec((1, emb), lambda p, i: (0, 0))
    return pl.pallas_call(
        functools.partial(_k23_body, n_nodes, n_graphs, half_rows, emb),
        grid=grid,
        in_specs=[
            row_spec, row_spec, row_spec, gid_spec,
            wd_spec, bd_spec, bd_spec,
            wd_spec, bd_spec, wd_spec, bd_spec, wd_spec, bd_spec, wd_spec,
            bd_spec,
            w_spec, b_spec, w_spec, b_spec, w_spec, b_spec, w_spec, b_spec,
        ],
        out_specs=pl.BlockSpec(memory_space=pltpu.SMEM),
        out_shape=jax.ShapeDtypeStruct((1, 1), jnp.float32),
        scratch_shapes=[
            pltpu.VMEM((half, twoemb), jnp.float32),
            pltpu.VMEM((n_graphs, emb), jnp.float32),
            pltpu.VMEM((n_graphs, emb), jnp.float32),
        ],
    )(p0, p1, y, gid2, W2d, b2d, b1d, lWad, lbad, lWbd, lbbd, lWcd, lbcd,
      lWsd, lbsd, gWa, gbar, gWb, gbbr, gWc, gbcr, gWs, gbsr)


def kernel(x, edge_index, graph_id, W1, b1, W2, b2, lWa, lba, lWb, lbb, lWc,
           lbc, lWs, lbs, gWa, gba, gWb, gbb, gWc, gbc, gWs, gbs):
    n_nodes, feat = x.shape
    emb = W1.shape[1]
    n_graphs = 128
    n_edges = edge_index.shape[1]

    block_rows = 2000
    k1_block = 5000

    n_chunks = n_edges // CHUNK
    ei3 = edge_index.reshape(2, n_chunks, CHUNK)

    y = _matmul_xw1(x, W1, k1_block)
    zeros_acc = jnp.zeros((n_nodes, emb), jnp.float32)
    partials = _sc_scatter(y, ei3, zeros_acc)

    half = n_nodes // 2
    y_p = y.reshape(half, 2 * emb)
    pp = partials.reshape(NC, half, 2 * emb)
    gid2 = graph_id.reshape(half, 2)

    def bd(w):
        z = jnp.zeros_like(w)
        return jnp.concatenate(
            [jnp.concatenate([w, z], 1), jnp.concatenate([z, w], 1)], 0)

    def b2x(b):
        return jnp.concatenate([b, b])[None, :]

    out = _k23(pp[0], pp[1], y_p, gid2, bd(W2), b2x(b2), b2x(b1),
               bd(lWa), b2x(lba), bd(lWb), b2x(lbb), bd(lWc), b2x(lbc),
               bd(lWs), b2x(lbs), gWa, gba[None, :], gWb, gbb[None, :],
               gWc, gbc[None, :], gWs, gbs[None, :], n_nodes, n_graphs,
               block_rows)
    return out[0, 0]
